# baseline (speedup 1.0000x reference)
import functools

import jax
import jax.numpy as jnp
from jax import lax
from jax.experimental import pallas as pl
from jax.experimental.pallas import tpu as pltpu

EMB = 128
BATCH = 16384
HIST = 200
N = BATCH * HIST
BR = 1024
G = N // BR


def _tc_body(idx_ref, hi_ref, lo_ref, out_ref):
  idx = idx_ref[0, 0]
  onehot = (idx[:, None] == lax.broadcasted_iota(jnp.int32, (1, 128), 1)
            ).astype(jnp.bfloat16)
  acc = jnp.dot(onehot, hi_ref[...], preferred_element_type=jnp.float32)
  acc = acc + jnp.dot(onehot, lo_ref[...], preferred_element_type=jnp.float32)
  out_ref[...] = acc


_tc_call = pl.pallas_call(
    _tc_body,
    grid=(G,),
    in_specs=[
        pl.BlockSpec((1, 1, BR), lambda g: (g, 0, 0)),
        pl.BlockSpec((128, EMB), lambda g: (0, 0)),
        pl.BlockSpec((128, EMB), lambda g: (0, 0)),
    ],
    out_specs=pl.BlockSpec((BR, EMB), lambda g: (g, 0)),
    out_shape=jax.ShapeDtypeStruct((N, EMB), jnp.float32),
)


def kernel(inputs, embeddings):
  idx = inputs.reshape(G, 1, BR).astype(jnp.int32)
  tpad = jnp.pad(embeddings, ((0, 128 - embeddings.shape[0]), (0, 0)))
  hi = tpad.astype(jnp.bfloat16)
  lo = (tpad - hi.astype(jnp.float32)).astype(jnp.bfloat16)
  out = _tc_call(idx, hi, lo)
  return out.reshape(BATCH, HIST, EMB)


# speedup vs baseline: 3.1932x; 3.1932x over previous
import functools

import jax
import jax.numpy as jnp
from jax import lax
from jax.experimental import pallas as pl
from jax.experimental.pallas import tpu as pltpu
from jax.experimental.pallas import tpu_sc as plsc

EMB = 128
BATCH = 16384
HIST = 200
N = BATCH * HIST
NC = 2
NS = 16
NW = NC * NS
PER_W = N // NW
SUB = 2
R = SUB * 128
NB = PER_W // R


def _make_sc_gather():
  mesh = plsc.VectorSubcoreMesh(core_axis_name="c", subcore_axis_name="s")

  @functools.partial(
      pl.kernel,
      mesh=mesh,
      out_type=jax.ShapeDtypeStruct((N, EMB), jnp.float32),
      scratch_types=[
          pltpu.VMEM((2, SUB, 128), jnp.int32),
          pltpu.VMEM((2, R, EMB), jnp.float32),
          pltpu.VMEM_SHARED((122, EMB), jnp.float32),
          pltpu.SemaphoreType.DMA,
          pltpu.SemaphoreType.DMA,
          pltpu.SemaphoreType.DMA,
          pltpu.SemaphoreType.DMA,
      ],
  )
  def k(idx_hbm, table_hbm, out_hbm, idx_v, rows_v, table_v, gs0, gs1, ws0, ws1):
    wid = lax.axis_index("s") * NC + lax.axis_index("c")
    base = wid * PER_W
    gs = (gs0, gs1)
    ws = (ws0, ws1)

    @pl.when(lax.axis_index("s") == 0)
    def _stage_table():
      pltpu.sync_copy(table_hbm, table_v)

    plsc.subcore_barrier()

    def load_idx(g, b):
      row0 = wid * (PER_W // 128) + g * SUB
      pltpu.sync_copy(idx_hbm.at[pl.ds(row0, SUB)], idx_v.at[b])

    def fire_gather(b):
      for s in range(SUB):
        pltpu.async_copy(
            table_v.at[idx_v.at[b, s]],
            rows_v.at[b, pl.ds(s * 128, 128)],
            gs[b],
        )

    def wait_gather(b):
      for s in range(SUB):
        pltpu.make_async_copy(
            table_v.at[idx_v.at[b, s]],
            rows_v.at[b, pl.ds(s * 128, 128)],
            gs[b],
        ).wait()

    def fire_write(g, b):
      off = base + g * R
      pltpu.async_copy(rows_v.at[b], out_hbm.at[pl.ds(off, R)], ws[b])

    def wait_write(b):
      pltpu.make_async_copy(
          rows_v.at[b], out_hbm.at[pl.ds(base, R)], ws[b]
      ).wait()

    load_idx(0, 0)
    fire_gather(0)

    def outer(j, carry):
      for b in (0, 1):
        g = 2 * j + b
        b2 = 1 - b

        @pl.when(g + 1 < NB)
        def _prep():
          load_idx(g + 1, b2)

          @pl.when(g >= 1)
          def _drain():
            wait_write(b2)

          fire_gather(b2)

        wait_gather(b)
        fire_write(g, b)
      return carry

    lax.fori_loop(0, NB // 2, outer, 0)
    wait_write(0)
    wait_write(1)

  return k


_sc_gather = _make_sc_gather()


def kernel(inputs, embeddings):
  idx = inputs.reshape(N // 128, 128).astype(jnp.int32)
  out = _sc_gather(idx, embeddings)
  return out.reshape(BATCH, HIST, EMB)


# speedup vs baseline: 3.1945x; 1.0004x over previous
import functools

import jax
import jax.numpy as jnp
from jax import lax
from jax.experimental import pallas as pl
from jax.experimental.pallas import tpu as pltpu
from jax.experimental.pallas import tpu_sc as plsc

EMB = 128
BATCH = 16384
HIST = 200
N = BATCH * HIST
NC = 2
NS = 16
NW = NC * NS
PER_W = N // NW
SUB = 2
R = SUB * 128
NB = PER_W // R
CH = 16
IDXROWS = CH * SUB
NCH = NB // CH


def _make_sc_gather():
  mesh = plsc.VectorSubcoreMesh(core_axis_name="c", subcore_axis_name="s")

  @functools.partial(
      pl.kernel,
      mesh=mesh,
      out_type=jax.ShapeDtypeStruct((N, EMB), jnp.float32),
      scratch_types=[
          pltpu.VMEM((2, IDXROWS, 128), jnp.int32),
          pltpu.VMEM((2, R, EMB), jnp.float32),
          pltpu.VMEM_SHARED((122, EMB), jnp.float32),
          pltpu.SemaphoreType.DMA,
          pltpu.SemaphoreType.DMA,
          pltpu.SemaphoreType.DMA,
          pltpu.SemaphoreType.DMA,
          pltpu.SemaphoreType.DMA,
          pltpu.SemaphoreType.DMA,
      ],
  )
  def k(idx_hbm, table_hbm, out_hbm, idx_v, rows_v, table_v,
        gs0, gs1, ws0, ws1, is0, is1):
    wid = lax.axis_index("s") * NC + lax.axis_index("c")
    base = wid * PER_W
    gs = (gs0, gs1)
    ws = (ws0, ws1)
    isem = (is0, is1)

    @pl.when(lax.axis_index("s") == 0)
    def _stage_table():
      pltpu.sync_copy(table_hbm, table_v)

    plsc.subcore_barrier()

    def fire_idx(c, cb):
      row0 = wid * (PER_W // 128) + c * IDXROWS
      pltpu.async_copy(
          idx_hbm.at[pl.ds(row0, IDXROWS)], idx_v.at[cb], isem[cb]
      )

    def wait_idx(cb):
      pltpu.make_async_copy(
          idx_hbm.at[pl.ds(0, IDXROWS)], idx_v.at[cb], isem[cb]
      ).wait()

    def fire_gather(g, b):
      cb = (g // CH) % 2
      p = (g % CH) * SUB
      for s in range(SUB):
        pltpu.async_copy(
            table_v.at[idx_v.at[cb, p + s]],
            rows_v.at[b, pl.ds(s * 128, 128)],
            gs[b],
        )

    def wait_gather(b):
      for s in range(SUB):
        pltpu.make_async_copy(
            table_v.at[idx_v.at[0, s]],
            rows_v.at[b, pl.ds(s * 128, 128)],
            gs[b],
        ).wait()

    def fire_write(g, b):
      off = base + g * R
      pltpu.async_copy(rows_v.at[b], out_hbm.at[pl.ds(off, R)], ws[b])

    def wait_write(b):
      pltpu.make_async_copy(
          rows_v.at[b], out_hbm.at[pl.ds(base, R)], ws[b]
      ).wait()

    fire_idx(0, 0)
    fire_idx(1, 1)
    wait_idx(0)
    fire_gather(0, 0)

    def outer(j, carry):
      for b in (0, 1):
        g = 2 * j + b
        b2 = 1 - b

        @pl.when(g + 1 < NB)
        def _prep():
          for slot in (0, 1):
            @pl.when(((g + 1) % CH == 0) & (((g + 1) // CH) % 2 == slot))
            def _idx_arrived(slot=slot):
              wait_idx(slot)

          @pl.when(g >= 1)
          def _drain():
            wait_write(b2)

          fire_gather(g + 1, b2)

        wait_gather(b)

        for slot in (0, 1):
          @pl.when((g % CH == CH - 1) & (g // CH + 2 < NCH)
                   & ((g // CH) % 2 == slot))
          def _prefetch(slot=slot):
            fire_idx(g // CH + 2, slot)

        fire_write(g, b)
      return carry

    lax.fori_loop(0, NB // 2, outer, 0)
    wait_write(0)
    wait_write(1)

  return k


_sc_gather = _make_sc_gather()


def kernel(inputs, embeddings):
  idx = inputs.reshape(N // 128, 128).astype(jnp.int32)
  out = _sc_gather(idx, embeddings)
  return out.reshape(BATCH, HIST, EMB)
